# baseline (speedup 1.0000x reference)
import functools

import jax
import jax.numpy as jnp
from jax import lax
from jax.experimental import pallas as pl
from jax.experimental.pallas import tpu as pltpu
from jax.experimental.pallas import tpu_sc as plsc

N = 10000
D = 128
E = 320000

NTILES = 16
NPAD = 10240
NPT = NPAD // NTILES
RCHUNK = 64
NRCH = NPT // RCHUNK
CPT = 160
EPT = CPT * 128
PAD_BASE = 10100


def _rsqrt_newton(d):
    y = jnp.minimum(0.5 * (d + 1.0), d * (1.0 / 32.0) + 16.0)
    for _ in range(8):
        y = 0.5 * (y + d / y)
    return 1.0 / y


EROWS = E // 128
ERLAST = EROWS - (NTILES - 1) * CPT


def _sag_kernel(x_hbm, rowm, colm, w_hbm, b_hbm, out_hbm, score_hbm,
                row2d, col2d, g2d, uloc, xbuf, xbuf2, obuf, obuf2, wbuf,
                bbuf, zbuf, ubuf, abuf, dbuf, sbuf, zerobuf, sp_u, sp_acc,
                ssem, xs1, xs2, os1, os2):
    s = lax.axis_index("s")
    nbase = s * NPT

    last = s == NTILES - 1

    lane = lax.iota(jnp.int32, 16)

    @pl.when(jnp.logical_not(last))
    def _efull():
        pltpu.sync_copy(rowm.at[pl.ds(s * CPT, CPT)], row2d)
        pltpu.sync_copy(colm.at[pl.ds(s * CPT, CPT)], col2d)

    @pl.when(last)
    def _etail():
        pltpu.sync_copy(rowm.at[pl.ds((NTILES - 1) * CPT, ERLAST)],
                        row2d.at[pl.ds(0, ERLAST)])
        pltpu.sync_copy(colm.at[pl.ds((NTILES - 1) * CPT, ERLAST)],
                        col2d.at[pl.ds(0, ERLAST)])
        padv = PAD_BASE + lane
        for rr in range(ERLAST, CPT):
            for k in range(8):
                row2d[rr, pl.ds(k * 16, 16)] = padv
                col2d[rr, pl.ds(k * 16, 16)] = padv

    pltpu.sync_copy(w_hbm, wbuf)
    pltpu.sync_copy(b_hbm, bbuf)

    zeros16 = jnp.zeros((16,), jnp.float32)

    def _zero(i, _):
        zerobuf[pl.ds(i * 16, 16)] = zeros16
        zbuf[pl.ds(i * 16, 16)] = zeros16
        return _
    lax.fori_loop(0, NPT // 16, _zero, None)
    pltpu.sync_copy(zerobuf, sp_acc.at[pl.ds(nbase, NPT)])

    nfull = jnp.where(last, (N - (NTILES - 1) * NPT) // RCHUNK, NRCH)

    ones16 = jnp.full((16,), 1.0, jnp.float32)
    for k in range(8):
        g2d[0, pl.ds(k * 16, 16)] = ones16

    plsc.subcore_barrier()

    def _deg_fire(j, _):
        pltpu.async_copy(g2d.at[0], sp_acc.at[col2d.at[j]], ssem, add=True)
        return _
    lax.fori_loop(0, CPT, _deg_fire, None)

    perms = [lane ^ sh for sh in (8, 4, 2, 1)]

    _gdn = lax.GatherDimensionNumbers(
        offset_dims=(), collapsed_slice_dims=(0,), start_index_map=(0,))

    def _hsum(v):
        for p in perms:
            v = v + lax.gather(v, p[:, None], _gdn, (1,),
                               mode=lax.GatherScatterMode.PROMISE_IN_BOUNDS)
        return v

    wv = [wbuf[pl.ds(k * 16, 16)] for k in range(8)]

    def _zgroup16(xb, rbase, zoff):
        def _zrow(r16, zv):
            r = rbase + r16
            vacc = xb[r, pl.ds(0, 16)] * wv[0]
            for k in range(1, 8):
                vacc = vacc + xb[r, pl.ds(k * 16, 16)] * wv[k]
            return jnp.where(lane == r16, _hsum(vacc), zv)
        zv = lax.fori_loop(0, 16, _zrow, zeros16)
        zbuf[pl.ds(zoff, 16)] = zv

    def _xin(j, xb, sem):
        return pltpu.async_copy(
            x_hbm.at[pl.ds(nbase + j * RCHUNK, RCHUNK)], xb, sem)

    def _xin_wait(j, xb, sem):
        pltpu.make_async_copy(
            x_hbm.at[pl.ds(nbase + j * RCHUNK, RCHUNK)], xb, sem).wait()

    _xin(0, xbuf, xs1)

    def _zchunk2(i, _):
        j0 = 2 * i
        _xin(j0 + 1, xbuf2, xs2)
        _xin_wait(j0, xbuf, xs1)

        def _zgroupa(g, _):
            _zgroup16(xbuf, g * 16, j0 * RCHUNK + g * 16)
            return _
        lax.fori_loop(0, RCHUNK // 16, _zgroupa, None)

        @pl.when(j0 + 2 < nfull)
        def _pref():
            _xin(j0 + 2, xbuf, xs1)
        _xin_wait(j0 + 1, xbuf2, xs2)

        def _zgroupb(g, _):
            _zgroup16(xbuf2, g * 16, (j0 + 1) * RCHUNK + g * 16)
            return _
        lax.fori_loop(0, RCHUNK // 16, _zgroupb, None)
        return _
    lax.fori_loop(0, nfull // 2, _zchunk2, None)

    @pl.when(last)
    def _ztail():
        pltpu.sync_copy(x_hbm.at[pl.ds(N - 16, 16)], xbuf.at[pl.ds(0, 16)])
        _zgroup16(xbuf, 0, N - 16 - nbase)

    def _deg_drain(j, _):
        pltpu.make_async_copy(g2d.at[0], sp_acc.at[col2d.at[j]], ssem).wait()
        return _
    lax.fori_loop(0, CPT, _deg_drain, None)
    plsc.subcore_barrier()

    pltpu.sync_copy(sp_acc.at[pl.ds(nbase, NPT)], abuf)

    def _dinv(i, _):
        sl = pl.ds(i * 16, 16)
        d = abuf[sl] + 1.0
        y = _rsqrt_newton(d)
        dbuf[sl] = y
        ubuf[sl] = y * zbuf[sl]
        return _
    lax.fori_loop(0, NPT // 16, _dinv, None)

    pltpu.sync_copy(zerobuf, sp_acc.at[pl.ds(nbase, NPT)])
    pltpu.sync_copy(ubuf, sp_u.at[pl.ds(nbase, NPT)])
    plsc.subcore_barrier()

    for step in range(2):
        pltpu.sync_copy(sp_u, uloc)

        def _gath(j, _):
            for k in range(8):
                iv = row2d[j, pl.ds(k * 16, 16)]
                g2d[j, pl.ds(k * 16, 16)] = plsc.load_gather(uloc, [iv])
            pltpu.async_copy(g2d.at[j], sp_acc.at[col2d.at[j]], ssem, add=True)
            return _
        lax.fori_loop(0, CPT, _gath, None)

        def _drain(j, _):
            pltpu.make_async_copy(g2d.at[j], sp_acc.at[col2d.at[j]], ssem).wait()
            return _
        lax.fori_loop(0, CPT, _drain, None)
        plsc.subcore_barrier()

        pltpu.sync_copy(sp_acc.at[pl.ds(nbase, NPT)], abuf)

        if step == 0:
            def _upd(i, _):
                sl = pl.ds(i * 16, 16)
                y = dbuf[sl]
                ubuf[sl] = y * y * (abuf[sl] + ubuf[sl])
                return _
            lax.fori_loop(0, NPT // 16, _upd, None)
            pltpu.sync_copy(zerobuf, sp_acc.at[pl.ds(nbase, NPT)])
            pltpu.sync_copy(ubuf, sp_u.at[pl.ds(nbase, NPT)])
            plsc.subcore_barrier()
        else:
            bval = bbuf[pl.ds(0, 16)][0]

            def _score(i, _):
                sl = pl.ds(i * 16, 16)
                logit = dbuf[sl] * (abuf[sl] + ubuf[sl]) + bval
                sbuf[sl] = 1.0 / (1.0 + jnp.exp(-logit))
                return _
            lax.fori_loop(0, NPT // 16, _score, None)

    @pl.when(jnp.logical_not(last))
    def _sfull():
        pltpu.sync_copy(sbuf, score_hbm.at[pl.ds(nbase, NPT)])

    @pl.when(last)
    def _stail():
        pltpu.sync_copy(sbuf.at[pl.ds(0, N - (NTILES - 1) * NPT)],
                        score_hbm.at[pl.ds((NTILES - 1) * NPT,
                                           N - (NTILES - 1) * NPT)])

    def _ogroup16(xb, ob, rbase, soff):
        sv16 = sbuf[pl.ds(soff, 16)]
        for r16 in range(16):
            r = rbase + r16
            sv = sv16[r16]
            for k in range(8):
                sl = pl.ds(k * 16, 16)
                ob[r, sl] = xb[r, sl] * sv

    def _oput(j, ob, sem):
        return pltpu.async_copy(
            ob, out_hbm.at[pl.ds(nbase + j * RCHUNK, RCHUNK)], sem)

    def _oput_wait(j, ob, sem):
        pltpu.make_async_copy(
            ob, out_hbm.at[pl.ds(nbase + j * RCHUNK, RCHUNK)], sem).wait()

    def _ocompute(j, xb, ob):
        def _ogroup(g, _):
            _ogroup16(xb, ob, g * 16, j * RCHUNK + g * 16)
            return _
        lax.fori_loop(0, RCHUNK // 16, _ogroup, None)

    _xin(0, xbuf, xs1)

    def _ochunk2(i, _):
        j0 = 2 * i
        _xin(j0 + 1, xbuf2, xs2)
        _xin_wait(j0, xbuf, xs1)

        @pl.when(i > 0)
        def _w1():
            _oput_wait(j0 - 2, obuf, os1)
        _ocompute(j0, xbuf, obuf)
        _oput(j0, obuf, os1)

        @pl.when(j0 + 2 < nfull)
        def _pref():
            _xin(j0 + 2, xbuf, xs1)
        _xin_wait(j0 + 1, xbuf2, xs2)

        @pl.when(i > 0)
        def _w2():
            _oput_wait(j0 - 1, obuf2, os2)
        _ocompute(j0 + 1, xbuf2, obuf2)
        _oput(j0 + 1, obuf2, os2)
        return _
    lax.fori_loop(0, nfull // 2, _ochunk2, None)

    _oput_wait(nfull - 2, obuf, os1)
    _oput_wait(nfull - 1, obuf2, os2)

    @pl.when(last)
    def _otail():
        pltpu.sync_copy(x_hbm.at[pl.ds(N - 16, 16)], xbuf.at[pl.ds(0, 16)])
        _ogroup16(xbuf, obuf, 0, N - 16 - nbase)
        pltpu.sync_copy(obuf.at[pl.ds(0, 16)], out_hbm.at[pl.ds(N - 16, 16)])


@jax.jit
def kernel(x, edge_index, W, b):
    rowm = edge_index[0].reshape(EROWS, 128)
    colm = edge_index[1].reshape(EROWS, 128)
    w = W.reshape(D)
    bp = jnp.pad(b, (0, 15))

    mesh = plsc.VectorSubcoreMesh(core_axis_name="c", subcore_axis_name="s",
                                  num_cores=1)
    run = pl.kernel(
        _sag_kernel,
        mesh=mesh,
        compiler_params=pltpu.CompilerParams(needs_layout_passes=False),
        out_type=[
            jax.ShapeDtypeStruct((N, D), jnp.float32),
            jax.ShapeDtypeStruct((N,), jnp.float32),
        ],
        scratch_types=[
            pltpu.VMEM((CPT, 128), jnp.int32),
            pltpu.VMEM((CPT, 128), jnp.int32),
            pltpu.VMEM((CPT, 128), jnp.float32),
            pltpu.VMEM((NPAD,), jnp.float32),
            pltpu.VMEM((RCHUNK, D), jnp.float32),
            pltpu.VMEM((RCHUNK, D), jnp.float32),
            pltpu.VMEM((RCHUNK, D), jnp.float32),
            pltpu.VMEM((RCHUNK, D), jnp.float32),
            pltpu.VMEM((D,), jnp.float32),
            pltpu.VMEM((16,), jnp.float32),
            pltpu.VMEM((NPT,), jnp.float32),
            pltpu.VMEM((NPT,), jnp.float32),
            pltpu.VMEM((NPT,), jnp.float32),
            pltpu.VMEM((NPT,), jnp.float32),
            pltpu.VMEM((NPT,), jnp.float32),
            pltpu.VMEM((NPT,), jnp.float32),
            pltpu.VMEM_SHARED((NPAD,), jnp.float32),
            pltpu.VMEM_SHARED((NPAD,), jnp.float32),
            pltpu.SemaphoreType.DMA,
            pltpu.SemaphoreType.DMA,
            pltpu.SemaphoreType.DMA,
            pltpu.SemaphoreType.DMA,
            pltpu.SemaphoreType.DMA,
        ],
    )
    out, score = run(x, rowm, colm, w, bp)
    return out, score


# speedup vs baseline: 1.0340x; 1.0340x over previous
import functools

import jax
import jax.numpy as jnp
from jax import lax
from jax.experimental import pallas as pl
from jax.experimental.pallas import tpu as pltpu
from jax.experimental.pallas import tpu_sc as plsc

N = 10000
D = 128
E = 320000

NTILES = 16
NPAD = 10240
NPT = NPAD // NTILES
RCHUNK = 64
NRCH = NPT // RCHUNK
CPT = 157
EPT = CPT * 128
EPAD = NTILES * EPT
PAD_BASE = 10100


def _rsqrt_newton(d):
    y = jnp.minimum(0.5 * (d + 1.0), d * (1.0 / 32.0) + 16.0)
    for _ in range(8):
        y = 0.5 * (y + d / y)
    return 1.0 / y


def _sag_kernel(x_hbm, row3d, col3d, w_hbm, b_hbm, out_hbm, score_hbm,
                row2d, col2d, g2d, uloc, xbuf, xbuf2, obuf, obuf2, wbuf,
                bbuf, zbuf, ubuf, abuf, dbuf, sbuf, zerobuf, sp_u, sp_acc,
                ssem, xs1, xs2, os1, os2):
    s = lax.axis_index("s")
    nbase = s * NPT

    pltpu.sync_copy(row3d.at[s], row2d)
    pltpu.sync_copy(col3d.at[s], col2d)
    pltpu.sync_copy(w_hbm, wbuf)
    pltpu.sync_copy(b_hbm, bbuf)

    zeros16 = jnp.zeros((16,), jnp.float32)

    def _zero(i, _):
        zerobuf[pl.ds(i * 16, 16)] = zeros16
        zbuf[pl.ds(i * 16, 16)] = zeros16
        return _
    lax.fori_loop(0, NPT // 16, _zero, None)
    pltpu.sync_copy(zerobuf, sp_acc.at[pl.ds(nbase, NPT)])

    last = s == NTILES - 1
    nfull = jnp.where(last, (N - (NTILES - 1) * NPT) // RCHUNK, NRCH)

    ones16 = jnp.full((16,), 1.0, jnp.float32)
    for k in range(8):
        g2d[0, pl.ds(k * 16, 16)] = ones16

    plsc.subcore_barrier()

    def _deg_fire(j, _):
        pltpu.async_copy(g2d.at[0], sp_acc.at[col2d.at[j]], ssem, add=True)
        return _
    lax.fori_loop(0, CPT, _deg_fire, None)

    lane = lax.iota(jnp.int32, 16)
    perms = [lane ^ sh for sh in (8, 4, 2, 1)]

    _gdn = lax.GatherDimensionNumbers(
        offset_dims=(), collapsed_slice_dims=(0,), start_index_map=(0,))

    def _hsum(v):
        for p in perms:
            v = v + lax.gather(v, p[:, None], _gdn, (1,),
                               mode=lax.GatherScatterMode.PROMISE_IN_BOUNDS)
        return v

    wv = [wbuf[pl.ds(k * 16, 16)] for k in range(8)]

    def _zgroup16(xb, rbase, zoff):
        def _zrow(r16, zv):
            r = rbase + r16
            vacc = xb[r, pl.ds(0, 16)] * wv[0]
            for k in range(1, 8):
                vacc = vacc + xb[r, pl.ds(k * 16, 16)] * wv[k]
            return jnp.where(lane == r16, _hsum(vacc), zv)
        zv = lax.fori_loop(0, 16, _zrow, zeros16)
        zbuf[pl.ds(zoff, 16)] = zv

    def _xin(j, xb, sem):
        return pltpu.async_copy(
            x_hbm.at[pl.ds(nbase + j * RCHUNK, RCHUNK)], xb, sem)

    def _xin_wait(j, xb, sem):
        pltpu.make_async_copy(
            x_hbm.at[pl.ds(nbase + j * RCHUNK, RCHUNK)], xb, sem).wait()

    _xin(0, xbuf, xs1)

    def _zchunk2(i, _):
        j0 = 2 * i
        _xin(j0 + 1, xbuf2, xs2)
        _xin_wait(j0, xbuf, xs1)

        def _zgroupa(g, _):
            _zgroup16(xbuf, g * 16, j0 * RCHUNK + g * 16)
            return _
        lax.fori_loop(0, RCHUNK // 16, _zgroupa, None)

        @pl.when(j0 + 2 < nfull)
        def _pref():
            _xin(j0 + 2, xbuf, xs1)
        _xin_wait(j0 + 1, xbuf2, xs2)

        def _zgroupb(g, _):
            _zgroup16(xbuf2, g * 16, (j0 + 1) * RCHUNK + g * 16)
            return _
        lax.fori_loop(0, RCHUNK // 16, _zgroupb, None)
        return _
    lax.fori_loop(0, nfull // 2, _zchunk2, None)

    @pl.when(last)
    def _ztail():
        pltpu.sync_copy(x_hbm.at[pl.ds(N - 16, 16)], xbuf.at[pl.ds(0, 16)])
        _zgroup16(xbuf, 0, N - 16 - nbase)

    def _deg_drain(j, _):
        pltpu.make_async_copy(g2d.at[0], sp_acc.at[col2d.at[j]], ssem).wait()
        return _
    lax.fori_loop(0, CPT, _deg_drain, None)
    plsc.subcore_barrier()

    pltpu.sync_copy(sp_acc.at[pl.ds(nbase, NPT)], abuf)

    def _dinv(i, _):
        sl = pl.ds(i * 16, 16)
        d = abuf[sl] + 1.0
        y = _rsqrt_newton(d)
        dbuf[sl] = y
        ubuf[sl] = y * zbuf[sl]
        return _
    lax.fori_loop(0, NPT // 16, _dinv, None)

    pltpu.sync_copy(zerobuf, sp_acc.at[pl.ds(nbase, NPT)])
    pltpu.sync_copy(ubuf, sp_u.at[pl.ds(nbase, NPT)])
    plsc.subcore_barrier()

    for step in range(2):
        pltpu.sync_copy(sp_u, uloc)

        def _gath(j, _):
            for k in range(8):
                iv = row2d[j, pl.ds(k * 16, 16)]
                g2d[j, pl.ds(k * 16, 16)] = plsc.load_gather(uloc, [iv])
            pltpu.async_copy(g2d.at[j], sp_acc.at[col2d.at[j]], ssem, add=True)
            return _
        lax.fori_loop(0, CPT, _gath, None)

        def _drain(j, _):
            pltpu.make_async_copy(g2d.at[j], sp_acc.at[col2d.at[j]], ssem).wait()
            return _
        lax.fori_loop(0, CPT, _drain, None)
        plsc.subcore_barrier()

        pltpu.sync_copy(sp_acc.at[pl.ds(nbase, NPT)], abuf)

        if step == 0:
            def _upd(i, _):
                sl = pl.ds(i * 16, 16)
                y = dbuf[sl]
                ubuf[sl] = y * y * (abuf[sl] + ubuf[sl])
                return _
            lax.fori_loop(0, NPT // 16, _upd, None)
            pltpu.sync_copy(zerobuf, sp_acc.at[pl.ds(nbase, NPT)])
            pltpu.sync_copy(ubuf, sp_u.at[pl.ds(nbase, NPT)])
            plsc.subcore_barrier()
        else:
            bval = bbuf[pl.ds(0, 16)][0]

            def _score(i, _):
                sl = pl.ds(i * 16, 16)
                logit = dbuf[sl] * (abuf[sl] + ubuf[sl]) + bval
                sbuf[sl] = 1.0 / (1.0 + jnp.exp(-logit))
                return _
            lax.fori_loop(0, NPT // 16, _score, None)

    @pl.when(jnp.logical_not(last))
    def _sfull():
        pltpu.sync_copy(sbuf, score_hbm.at[pl.ds(nbase, NPT)])

    @pl.when(last)
    def _stail():
        pltpu.sync_copy(sbuf.at[pl.ds(0, N - (NTILES - 1) * NPT)],
                        score_hbm.at[pl.ds((NTILES - 1) * NPT,
                                           N - (NTILES - 1) * NPT)])

    def _ogroup16(xb, ob, rbase, soff):
        sv16 = sbuf[pl.ds(soff, 16)]
        for r16 in range(16):
            r = rbase + r16
            sv = sv16[r16]
            for k in range(8):
                sl = pl.ds(k * 16, 16)
                ob[r, sl] = xb[r, sl] * sv

    def _oput(j, ob, sem):
        return pltpu.async_copy(
            ob, out_hbm.at[pl.ds(nbase + j * RCHUNK, RCHUNK)], sem)

    def _oput_wait(j, ob, sem):
        pltpu.make_async_copy(
            ob, out_hbm.at[pl.ds(nbase + j * RCHUNK, RCHUNK)], sem).wait()

    def _ocompute(j, xb, ob):
        def _ogroup(g, _):
            _ogroup16(xb, ob, g * 16, j * RCHUNK + g * 16)
            return _
        lax.fori_loop(0, RCHUNK // 16, _ogroup, None)

    _xin(0, xbuf, xs1)

    def _ochunk2(i, _):
        j0 = 2 * i
        _xin(j0 + 1, xbuf2, xs2)
        _xin_wait(j0, xbuf, xs1)

        @pl.when(i > 0)
        def _w1():
            _oput_wait(j0 - 2, obuf, os1)
        _ocompute(j0, xbuf, obuf)
        _oput(j0, obuf, os1)

        @pl.when(j0 + 2 < nfull)
        def _pref():
            _xin(j0 + 2, xbuf, xs1)
        _xin_wait(j0 + 1, xbuf2, xs2)

        @pl.when(i > 0)
        def _w2():
            _oput_wait(j0 - 1, obuf2, os2)
        _ocompute(j0 + 1, xbuf2, obuf2)
        _oput(j0 + 1, obuf2, os2)
        return _
    lax.fori_loop(0, nfull // 2, _ochunk2, None)

    _oput_wait(nfull - 2, obuf, os1)
    _oput_wait(nfull - 1, obuf2, os2)

    @pl.when(last)
    def _otail():
        pltpu.sync_copy(x_hbm.at[pl.ds(N - 16, 16)], xbuf.at[pl.ds(0, 16)])
        _ogroup16(xbuf, obuf, 0, N - 16 - nbase)
        pltpu.sync_copy(obuf.at[pl.ds(0, 16)], out_hbm.at[pl.ds(N - 16, 16)])


@jax.jit
def kernel(x, edge_index, W, b):
    npad_e = EPAD - E
    pad_idx = PAD_BASE + (jnp.arange(npad_e, dtype=jnp.int32) % 64)
    row3d = jnp.concatenate([edge_index[0], pad_idx]).reshape(NTILES, CPT, 128)
    col3d = jnp.concatenate([edge_index[1], pad_idx]).reshape(NTILES, CPT, 128)
    w = W.reshape(D)
    bp = jnp.pad(b, (0, 15))

    mesh = plsc.VectorSubcoreMesh(core_axis_name="c", subcore_axis_name="s",
                                  num_cores=1)
    run = pl.kernel(
        _sag_kernel,
        mesh=mesh,
        compiler_params=pltpu.CompilerParams(needs_layout_passes=False),
        out_type=[
            jax.ShapeDtypeStruct((N, D), jnp.float32),
            jax.ShapeDtypeStruct((N,), jnp.float32),
        ],
        scratch_types=[
            pltpu.VMEM((CPT, 128), jnp.int32),
            pltpu.VMEM((CPT, 128), jnp.int32),
            pltpu.VMEM((CPT, 128), jnp.float32),
            pltpu.VMEM((NPAD,), jnp.float32),
            pltpu.VMEM((RCHUNK, D), jnp.float32),
            pltpu.VMEM((RCHUNK, D), jnp.float32),
            pltpu.VMEM((RCHUNK, D), jnp.float32),
            pltpu.VMEM((RCHUNK, D), jnp.float32),
            pltpu.VMEM((D,), jnp.float32),
            pltpu.VMEM((16,), jnp.float32),
            pltpu.VMEM((NPT,), jnp.float32),
            pltpu.VMEM((NPT,), jnp.float32),
            pltpu.VMEM((NPT,), jnp.float32),
            pltpu.VMEM((NPT,), jnp.float32),
            pltpu.VMEM((NPT,), jnp.float32),
            pltpu.VMEM((NPT,), jnp.float32),
            pltpu.VMEM_SHARED((NPAD,), jnp.float32),
            pltpu.VMEM_SHARED((NPAD,), jnp.float32),
            pltpu.SemaphoreType.DMA,
            pltpu.SemaphoreType.DMA,
            pltpu.SemaphoreType.DMA,
            pltpu.SemaphoreType.DMA,
            pltpu.SemaphoreType.DMA,
        ],
    )
    out, score = run(x, row3d, col3d, w, bp)
    return out, score
